# baseline (speedup 1.0000x reference)
import functools
import jax
import jax.numpy as jnp
from jax import lax
from jax.experimental import pallas as pl
from jax.experimental.pallas import tpu as pltpu
from jax.experimental.pallas import tpu_sc as plsc

VOC = 1_000_000
D = 64
SCALE = 8.0

NC = 2
NS = 16
NW = NC * NS
NB = 16384
NSEQ = 50
BPW = NB // NW
BT = BPW // 128
IW = 128
NSTREAM = BPW // IW
NJT = D // 8
HJT = NJT // 2

_mesh = plsc.VectorSubcoreMesh(core_axis_name="c", subcore_axis_name="s")


@functools.partial(
    pl.kernel,
    mesh=_mesh,
    out_type=jax.ShapeDtypeStruct((NSEQ, NJT, 128, 8, 128), jnp.float32),
    compiler_params=pltpu.CompilerParams(
        use_tc_tiling_on_sc=False, needs_layout_passes=False
    ),
    scratch_types=[
        pltpu.VMEM((NSTREAM, IW), jnp.int32),
        pltpu.VMEM((BPW, D), jnp.float32),
        pltpu.VMEM((BPW * (D + 1),), jnp.float32),
        pltpu.VMEM((2, HJT, BT, 8, 128), jnp.float32),
        pltpu.SemaphoreType.DMA,
        pltpu.SemaphoreType.DMA,
        pltpu.SemaphoreType.DMA,
    ],
)
def _emb_lookup(
    xt_hbm, tab_hbm, out_hbm, idx_v, rows_v, pad_v, seg_v, gsem, s0, s1
):
    wid = lax.axis_index("s") * NC + lax.axis_index("c")
    ssem = (s0, s1)

    lane = lax.iota(jnp.int32, 16)
    lane65 = lane * (D + 1)

    def load_idx_fire_gathers(s):
        pltpu.sync_copy(xt_hbm.at[s, pl.ds(wid * NSTREAM, NSTREAM)], idx_v)
        for j in range(NSTREAM):
            pltpu.async_copy(
                tab_hbm.at[idx_v.at[j]],
                rows_v.at[pl.ds(j * IW, IW)],
                gsem,
            )

    def wait_gathers():
        for j in range(NSTREAM):
            pltpu.make_async_copy(
                tab_hbm.at[pl.ds(0, IW)],
                rows_v.at[pl.ds(j * IW, IW)],
                gsem,
            ).wait()

    def copy_pass():
        @plsc.parallel_loop(0, BPW, unroll=4)
        def _copy_body(b):
            off = b * (D + 1)
            for jb in range(D // 16):
                pad_v[pl.ds(off + jb * 16, 16)] = rows_v[b, pl.ds(jb * 16, 16)]

    def fire_seg(s, half):
        for jl in range(HJT):
            pltpu.async_copy(
                seg_v.at[half, jl],
                out_hbm.at[s, half * HJT + jl, pl.ds(wid * BT, BT)],
                ssem[half],
            )

    def wait_seg(half):
        for jl in range(HJT):
            pltpu.make_async_copy(
                out_hbm.at[0, jl, pl.ds(wid * BT, BT)],
                seg_v.at[half, jl],
                ssem[half],
            ).wait()

    def transpose_half(half):
        for jl in range(HJT):
            jt = half * HJT + jl

            @plsc.parallel_loop(0, BT * 8, unroll=4)
            def _transpose_body(k):
                b0 = (k >> 3) * 128 + (k & 7) * 16
                base = b0 * (D + 1) + jt * 8
                for j8 in range(8):
                    idx = lane65 + (base + j8)
                    v = plsc.load_gather(pad_v, [idx])
                    seg_v[half, jl, k >> 3, j8, pl.ds((k & 7) * 16, 16)] = (
                        v * SCALE
                    )

    def step(s, fire_next):
        wait_gathers()
        copy_pass()
        if fire_next:
            load_idx_fire_gathers(s + 1)
        for half in range(2):
            wait_seg(half)
            transpose_half(half)
            fire_seg(s, half)

    for half in range(2):
        for jl in range(HJT):
            pltpu.async_copy(
                out_hbm.at[0, jl, pl.ds(wid * BT, BT)],
                seg_v.at[half, jl],
                ssem[half],
            )
    load_idx_fire_gathers(0)

    def loop_body(s, carry):
        step(s, True)
        return carry

    lax.fori_loop(0, NSEQ - 1, loop_body, 0)
    step(NSEQ - 1, False)
    wait_seg(0)
    wait_seg(1)


def kernel(x, table):
    xt = x.T.reshape(NSEQ, NB // IW, IW).astype(jnp.int32)
    out5 = _emb_lookup(xt, table)
    out = jnp.transpose(out5, (2, 4, 0, 1, 3)).reshape(NB, NSEQ, D)
    return out


# speedup vs baseline: 1.0633x; 1.0633x over previous
import functools
import jax
import jax.numpy as jnp
from jax import lax
from jax.experimental import pallas as pl
from jax.experimental.pallas import tpu as pltpu
from jax.experimental.pallas import tpu_sc as plsc

VOC = 1_000_000
D = 64
SCALE = 8.0

NC = 2
NS = 16
NW = NC * NS
NB = 16384
NSEQ = 50
BPW = NB // NW
BT = BPW // 128
IW = 128
NSTREAM = BPW // IW
NJT = D // 8
HJT = NJT // 2

_mesh = plsc.VectorSubcoreMesh(core_axis_name="c", subcore_axis_name="s")


@functools.partial(
    pl.kernel,
    mesh=_mesh,
    out_type=jax.ShapeDtypeStruct((NSEQ, NJT, 128, 8, 128), jnp.float32),
    compiler_params=pltpu.CompilerParams(
        use_tc_tiling_on_sc=False, needs_layout_passes=False
    ),
    scratch_types=[
        pltpu.VMEM((NSTREAM, IW), jnp.int32),
        pltpu.VMEM((BPW, D), jnp.float32),
        pltpu.VMEM((BPW * (D + 1),), jnp.float32),
        pltpu.VMEM((2, HJT, BT, 8, 128), jnp.float32),
        pltpu.SemaphoreType.DMA,
        pltpu.SemaphoreType.DMA,
        pltpu.SemaphoreType.DMA,
    ],
)
def _emb_lookup(
    xt_hbm, tab_hbm, out_hbm, idx_v, rows_v, pad_v, seg_v, gsem, s0, s1
):
    wid = lax.axis_index("s") * NC + lax.axis_index("c")
    ssem = (s0, s1)

    lane = lax.iota(jnp.int32, 16)
    lane65 = lane * (D + 1)

    def load_idx_fire_gathers(s):
        pltpu.sync_copy(xt_hbm.at[s, pl.ds(wid * NSTREAM, NSTREAM)], idx_v)
        for j in range(NSTREAM):
            pltpu.async_copy(
                tab_hbm.at[idx_v.at[j]],
                rows_v.at[pl.ds(j * IW, IW)],
                gsem,
            )

    def wait_gathers():
        for j in range(NSTREAM):
            pltpu.make_async_copy(
                tab_hbm.at[pl.ds(0, IW)],
                rows_v.at[pl.ds(j * IW, IW)],
                gsem,
            ).wait()

    def copy_pass():
        @plsc.parallel_loop(0, BPW, unroll=4)
        def _copy_body(b):
            off = b * (D + 1)
            for jb in range(D // 16):
                pad_v[pl.ds(off + jb * 16, 16)] = rows_v[b, pl.ds(jb * 16, 16)]

    def fire_seg(s, half):
        for jl in range(HJT):
            pltpu.async_copy(
                seg_v.at[half, jl],
                out_hbm.at[s, half * HJT + jl, pl.ds(wid * BT, BT)],
                ssem[half],
            )

    def wait_seg(half):
        for jl in range(HJT):
            pltpu.make_async_copy(
                out_hbm.at[0, jl, pl.ds(wid * BT, BT)],
                seg_v.at[half, jl],
                ssem[half],
            ).wait()

    def transpose_half(half):
        for jl in range(HJT):
            jt = half * HJT + jl

            @plsc.parallel_loop(0, BT * 8, unroll=2)
            def _transpose_body(k):
                b0 = (k >> 3) * 128 + (k & 7) * 16
                base = b0 * (D + 1) + jt * 8
                for j8 in range(8):
                    idx = lane65 + (base + j8)
                    v = plsc.load_gather(pad_v, [idx])
                    seg_v[half, jl, k >> 3, j8, pl.ds((k & 7) * 16, 16)] = (
                        v * SCALE
                    )

    def step(s, fire_next):
        wait_gathers()
        copy_pass()
        if fire_next:
            load_idx_fire_gathers(s + 1)
        for half in range(2):
            wait_seg(half)
            transpose_half(half)
            fire_seg(s, half)

    for half in range(2):
        for jl in range(HJT):
            pltpu.async_copy(
                out_hbm.at[0, jl, pl.ds(wid * BT, BT)],
                seg_v.at[half, jl],
                ssem[half],
            )
    load_idx_fire_gathers(0)

    def loop_body(s, carry):
        step(s, True)
        return carry

    lax.fori_loop(0, NSEQ - 1, loop_body, 0)
    step(NSEQ - 1, False)
    wait_seg(0)
    wait_seg(1)


def kernel(x, table):
    xt = x.T.reshape(NSEQ, NB // IW, IW).astype(jnp.int32)
    out5 = _emb_lookup(xt, table)
    out = jnp.transpose(out5, (2, 4, 0, 1, 3)).reshape(NB, NSEQ, D)
    return out
